# baseline (speedup 1.0000x reference)
import functools

import numpy as np

import jax
import jax.numpy as jnp
from jax import lax
from jax.experimental import pallas as pl
from jax.experimental.pallas import tpu as pltpu
from jax.experimental.pallas import tpu_sc as plsc

B = 16384
NSLOT = 6
SLOTS = B * NSLOT
NC, NS = 2, 16
NW = NC * NS
RPS = 8
Q = RPS * NSLOT
SPW = SLOTS // NW
BPW = B // NW
NQ = BPW // RPS
ROW = 120
MROWS = 12544
IROW0 = MROWS
SROWS = 12672
PROWS = 25000
TROW0 = PROWS + SROWS
TBLROWS = TROW0 + 2

_mesh = plsc.VectorSubcoreMesh(
    core_axis_name="c", subcore_axis_name="s", num_cores=NC, num_subcores=NS
)


@functools.partial(
    pl.kernel,
    out_type=jax.ShapeDtypeStruct((SLOTS * ROW,), jnp.float32),
    mesh=_mesh,
    scratch_types=[
        pltpu.VMEM_SHARED((SROWS, 128), jnp.float32),
        pltpu.VMEM((2, 128), jnp.float32),
        pltpu.VMEM((2 * Q,), jnp.int32),
        pltpu.VMEM((2 * 4 * Q,), jnp.int32),
        pltpu.VMEM((2 * Q,), jnp.int32),
        pltpu.VMEM((2 * Q,), jnp.int32),
        pltpu.VMEM((2, Q), jnp.int32),
        pltpu.VMEM((2, Q), jnp.int32),
        pltpu.VMEM((2, 4 * Q), jnp.int32),
        pltpu.VMEM((2, 4 * Q), jnp.int32),
        pltpu.VMEM((2, Q), jnp.int32),
        pltpu.VMEM((2, Q), jnp.int32),
        pltpu.VMEM((Q, 128), jnp.float32),
        pltpu.VMEM((4 * 16, 128), jnp.float32),
        pltpu.VMEM((2 * Q * ROW,), jnp.float32),
        pltpu.SemaphoreType.DMA,
        pltpu.SemaphoreType.DMA,
        pltpu.SemaphoreType.DMA,
        pltpu.SemaphoreType.DMA,
        pltpu.SemaphoreType.DMA,
        pltpu.SemaphoreType.DMA,
        pltpu.SemaphoreType.DMA,
        pltpu.SemaphoreType.DMA,
        pltpu.SemaphoreType.DMA,
    ],
    compiler_params=pltpu.CompilerParams(needs_layout_passes=False),
)
def _encode(ids_h, tbl_h, out_h,
            msp, ttab, pg, mg, ig, tg, pq, pr, mq, mr, iq, ir,
            pbuf, mbuf, obuf,
            isem0, isem1, psem, msem0, msem1, msem2, msem3,
            wsem0, wsem1):
    sid = lax.axis_index("s")
    cid = lax.axis_index("c")
    wid = sid * NC + cid
    isems = (isem0, isem1)
    msems = (msem0, msem1, msem2, msem3)
    wsems = (wsem0, wsem1)

    srows = SROWS // NS
    pltpu.sync_copy(tbl_h.at[pl.ds(PROWS + sid * srows, srows)],
                    msp.at[pl.ds(sid * srows, srows)])
    pltpu.sync_copy(tbl_h.at[pl.ds(TROW0, 2)], ttab)
    plsc.subcore_barrier()

    iota = lax.iota(jnp.int32, 16)

    def slot_base(c):
        return (iota + 16 * c) * ROW

    def move_base(fm0):
        fm = iota + fm0
        return (lax.shift_right_logical(fm, 2) * ROW + 32
                + lax.bitwise_and(fm, 3) * 16)

    def idx_list(k, b):
        base = wid * SPW + k * Q
        return [
            (ids_h.at[pl.ds(base, Q)], pg.at[pl.ds(b * Q, Q)]),
            (ids_h.at[pl.ds(SLOTS + 4 * base, 4 * Q)],
             mg.at[pl.ds(b * 4 * Q, 4 * Q)]),
            (ids_h.at[pl.ds(5 * SLOTS + base, Q)], ig.at[pl.ds(b * Q, Q)]),
            (ids_h.at[pl.ds(6 * SLOTS + base, Q)], tg.at[pl.ds(b * Q, Q)]),
        ]

    def stage_idx(k, b):
        for src, dst in idx_list(k, b):
            pltpu.async_copy(src, dst, isems[b])

    def wait_idx(k, b):
        for src, dst in idx_list(k, b):
            pltpu.make_async_copy(src, dst, isems[b]).wait()

    def splits(b):
        for c in range(Q // 16):
            sl = pl.ds(16 * c, 16)
            pv = pg[pl.ds(b * Q + 16 * c, 16)]
            pq[b, sl] = lax.shift_right_logical(pv, 2)
            pr[b, sl] = lax.bitwise_and(pv, 3)
            iv = ig[pl.ds(b * Q + 16 * c, 16)]
            iq[b, sl] = IROW0 + lax.shift_right_logical(iv, 3)
            ir[b, sl] = lax.bitwise_and(iv, 7)
        for c in range(4 * Q // 16):
            sl = pl.ds(16 * c, 16)
            mv = mg[pl.ds(b * 4 * Q + 16 * c, 16)]
            mq[b, sl] = lax.shift_right_logical(mv, 3)
            mr[b, sl] = lax.bitwise_and(mv, 7)

    def out_rows(k):
        return out_h.at[pl.ds((wid * SPW + k * Q) * ROW, Q * ROW)]

    def oscatter(b, offv, vals):
        plsc.store_scatter(obuf, [offv + b * Q * ROW], vals)

    stage_idx(0, 0)

    @pl.loop(0, NQ, step=2)
    def _(k0):
        for bb in range(2):
            k = k0 + bb
            b, b2 = bb, 1 - bb

            wait_idx(k, b)
            splits(b)

            @pl.when(k + 1 < NQ)
            def _():
                stage_idx(k + 1, b2)

            pltpu.async_copy(tbl_h.at[pq.at[b]], pbuf, psem)
            for s0 in range(3):
                pltpu.async_copy(msp.at[mq.at[b, pl.ds(16 * s0, 16)]],
                                 mbuf.at[pl.ds(16 * s0, 16)], msems[s0])

            @pl.when(k >= 2)
            def _():
                pltpu.make_async_copy(
                    obuf.at[pl.ds(b * Q * ROW, Q * ROW)], out_rows(k - 2),
                    wsems[b]).wait()

            for s in range(12):
                mb = s % 4
                if s + 3 < 12:
                    pltpu.async_copy(
                        msp.at[mq.at[b, pl.ds(16 * (s + 3), 16)]],
                        mbuf.at[pl.ds(((s + 3) % 4) * 16, 16)],
                        msems[(s + 3) % 4])
                pltpu.make_async_copy(
                    msp.at[mq.at[b, pl.ds(16 * s, 16)]],
                    mbuf.at[pl.ds(mb * 16, 16)], msems[mb]).wait()
                srows_v = iota + mb * 16
                offb = move_base(16 * s)
                col0 = mr[b, pl.ds(16 * s, 16)] * 16

                @pl.loop(0, 16, unroll=4)
                def _(d):
                    vals = plsc.load_gather(mbuf, [srows_v, col0 + d])
                    oscatter(b, offb + d, vals)

            pltpu.make_async_copy(tbl_h.at[pq.at[b]], pbuf, psem).wait()
            for c in range(3):
                srows_v = iota + 16 * c
                offb = slot_base(c)
                col0 = pr[b, pl.ds(16 * c, 16)] * 32

                @pl.loop(0, 32, unroll=4)
                def _(d):
                    vals = plsc.load_gather(pbuf, [srows_v, col0 + d])
                    oscatter(b, offb + d, vals)

            pltpu.async_copy(msp.at[iq.at[b]], pbuf.at[pl.ds(0, Q)], psem)

            for c in range(3):
                offb = slot_base(c) + 112
                tflat = tg[pl.ds(b * Q + 16 * c, 16)] * 8

                @pl.loop(0, 8, unroll=4)
                def _(d):
                    fl = tflat + d
                    vals = plsc.load_gather(
                        ttab, [lax.shift_right_logical(fl, 7),
                               lax.bitwise_and(fl, 127)])
                    oscatter(b, offb + d, vals)

            pltpu.make_async_copy(msp.at[iq.at[b]], pbuf.at[pl.ds(0, Q)],
                                  psem).wait()
            for c in range(3):
                srows_v = iota + 16 * c
                offb = slot_base(c) + 96
                col0 = ir[b, pl.ds(16 * c, 16)] * 16

                @pl.loop(0, 16, unroll=4)
                def _(d):
                    vals = plsc.load_gather(pbuf, [srows_v, col0 + d])
                    oscatter(b, offb + d, vals)

            pltpu.async_copy(obuf.at[pl.ds(b * Q * ROW, Q * ROW)],
                             out_rows(k), wsems[b])

    pltpu.make_async_copy(obuf.at[pl.ds(0, Q * ROW)], out_rows(NQ - 2),
                          wsems[0]).wait()
    pltpu.make_async_copy(obuf.at[pl.ds(Q * ROW, Q * ROW)],
                          out_rows(NQ - 1), wsems[1]).wait()


def kernel(pokemon_ids, move_ids, item_ids, tera_ids, P, M, I, T):
    ids = jnp.concatenate([
        pokemon_ids.astype(jnp.int32).reshape(SLOTS),
        move_ids.astype(jnp.int32).reshape(SLOTS * 4),
        item_ids.astype(jnp.int32).reshape(SLOTS),
        tera_ids.astype(jnp.int32).reshape(SLOTS),
    ])
    tbl = jnp.concatenate([
        P.reshape(PROWS, 128),
        jnp.pad(M.reshape(12500, 128), ((0, MROWS - 12500), (0, 0))),
        jnp.pad(I.reshape(125, 128), ((0, SROWS - IROW0 - 125), (0, 0))),
        jnp.pad(T.reshape(1, 160), ((0, 0), (0, 96))).reshape(2, 128),
    ])
    return _encode(ids, tbl).reshape(B, NSLOT * ROW)


# speedup vs baseline: 1.0445x; 1.0445x over previous
import functools

import numpy as np

import jax
import jax.numpy as jnp
from jax import lax
from jax.experimental import pallas as pl
from jax.experimental.pallas import tpu as pltpu
from jax.experimental.pallas import tpu_sc as plsc

B = 16384
NSLOT = 6
SLOTS = B * NSLOT
NC, NS = 2, 16
NW = NC * NS
RPS = 8
Q = RPS * NSLOT
SPW = SLOTS // NW
BPW = B // NW
NQ = BPW // RPS
ROW = 120
MROWS = 12544
IROW0 = MROWS
SROWS = 12672
PROWS = 25000
TROW0 = PROWS + SROWS
TBLROWS = TROW0 + 2

_mesh = plsc.VectorSubcoreMesh(
    core_axis_name="c", subcore_axis_name="s", num_cores=NC, num_subcores=NS
)


@functools.partial(
    pl.kernel,
    out_type=jax.ShapeDtypeStruct((B, NSLOT * ROW), jnp.float32),
    mesh=_mesh,
    scratch_types=[
        pltpu.VMEM_SHARED((SROWS, 128), jnp.float32),
        pltpu.VMEM((2, 128), jnp.float32),
        pltpu.VMEM((2 * Q,), jnp.int32),
        pltpu.VMEM((2 * 4 * Q,), jnp.int32),
        pltpu.VMEM((2 * Q,), jnp.int32),
        pltpu.VMEM((2 * Q,), jnp.int32),
        pltpu.VMEM((2, Q), jnp.int32),
        pltpu.VMEM((2, Q), jnp.int32),
        pltpu.VMEM((2, 4 * Q), jnp.int32),
        pltpu.VMEM((2, 4 * Q), jnp.int32),
        pltpu.VMEM((2, Q), jnp.int32),
        pltpu.VMEM((2, Q), jnp.int32),
        pltpu.VMEM((Q, 128), jnp.float32),
        pltpu.VMEM((4 * 16, 128), jnp.float32),
        pltpu.VMEM((2 * RPS, NSLOT * ROW), jnp.float32),
        pltpu.SemaphoreType.DMA,
        pltpu.SemaphoreType.DMA,
        pltpu.SemaphoreType.DMA,
        pltpu.SemaphoreType.DMA,
        pltpu.SemaphoreType.DMA,
        pltpu.SemaphoreType.DMA,
        pltpu.SemaphoreType.DMA,
        pltpu.SemaphoreType.DMA,
        pltpu.SemaphoreType.DMA,
    ],
    compiler_params=pltpu.CompilerParams(needs_layout_passes=False),
)
def _encode(ids_h, tbl_h, out_h,
            msp, ttab, pg, mg, ig, tg, pq, pr, mq, mr, iq, ir,
            pbuf, mbuf, obuf,
            isem0, isem1, psem, msem0, msem1, msem2, msem3,
            wsem0, wsem1):
    sid = lax.axis_index("s")
    cid = lax.axis_index("c")
    wid = sid * NC + cid
    isems = (isem0, isem1)
    msems = (msem0, msem1, msem2, msem3)
    wsems = (wsem0, wsem1)

    srows = SROWS // NS
    pltpu.sync_copy(tbl_h.at[pl.ds(PROWS + sid * srows, srows)],
                    msp.at[pl.ds(sid * srows, srows)])
    pltpu.sync_copy(tbl_h.at[pl.ds(TROW0, 2)], ttab)
    plsc.subcore_barrier()

    iota = lax.iota(jnp.int32, 16)

    def div6(v):
        return lax.shift_right_logical(v * 43691, 18)

    def slot_rowcol(c):
        l = iota + 16 * c
        lb = div6(l)
        return lb, (l - lb * NSLOT) * ROW

    def move_rowcol(fm0):
        fm = iota + fm0
        ml = lax.shift_right_logical(fm, 2)
        mlb = div6(ml)
        return mlb, ((ml - mlb * NSLOT) * ROW + 32
                     + lax.bitwise_and(fm, 3) * 16)

    def idx_list(k, b):
        base = wid * SPW + k * Q
        return [
            (ids_h.at[pl.ds(base, Q)], pg.at[pl.ds(b * Q, Q)]),
            (ids_h.at[pl.ds(SLOTS + 4 * base, 4 * Q)],
             mg.at[pl.ds(b * 4 * Q, 4 * Q)]),
            (ids_h.at[pl.ds(5 * SLOTS + base, Q)], ig.at[pl.ds(b * Q, Q)]),
            (ids_h.at[pl.ds(6 * SLOTS + base, Q)], tg.at[pl.ds(b * Q, Q)]),
        ]

    def stage_idx(k, b):
        for src, dst in idx_list(k, b):
            pltpu.async_copy(src, dst, isems[b])

    def wait_idx(k, b):
        for src, dst in idx_list(k, b):
            pltpu.make_async_copy(src, dst, isems[b]).wait()

    def splits(b):
        for c in range(Q // 16):
            sl = pl.ds(16 * c, 16)
            pv = pg[pl.ds(b * Q + 16 * c, 16)]
            pq[b, sl] = lax.shift_right_logical(pv, 2)
            pr[b, sl] = lax.bitwise_and(pv, 3)
            iv = ig[pl.ds(b * Q + 16 * c, 16)]
            iq[b, sl] = IROW0 + lax.shift_right_logical(iv, 3)
            ir[b, sl] = lax.bitwise_and(iv, 7)
        for c in range(4 * Q // 16):
            sl = pl.ds(16 * c, 16)
            mv = mg[pl.ds(b * 4 * Q + 16 * c, 16)]
            mq[b, sl] = lax.shift_right_logical(mv, 3)
            mr[b, sl] = lax.bitwise_and(mv, 7)

    def out_rows(k):
        return out_h.at[pl.ds(wid * BPW + k * RPS, RPS)]

    def oscatter(b, rowv, colv, vals):
        plsc.store_scatter(obuf, [rowv + b * RPS, colv], vals)

    stage_idx(0, 0)

    @pl.loop(0, NQ, step=2)
    def _(k0):
        for bb in range(2):
            k = k0 + bb
            b, b2 = bb, 1 - bb

            wait_idx(k, b)
            splits(b)

            @pl.when(k + 1 < NQ)
            def _():
                stage_idx(k + 1, b2)

            pltpu.async_copy(tbl_h.at[pq.at[b]], pbuf, psem)
            for s0 in range(3):
                pltpu.async_copy(msp.at[mq.at[b, pl.ds(16 * s0, 16)]],
                                 mbuf.at[pl.ds(16 * s0, 16)], msems[s0])

            @pl.when(k >= 2)
            def _():
                pltpu.make_async_copy(
                    obuf.at[pl.ds(b * RPS, RPS)], out_rows(k - 2),
                    wsems[b]).wait()

            for s in range(12):
                mb = s % 4
                if s + 3 < 12:
                    pltpu.async_copy(
                        msp.at[mq.at[b, pl.ds(16 * (s + 3), 16)]],
                        mbuf.at[pl.ds(((s + 3) % 4) * 16, 16)],
                        msems[(s + 3) % 4])
                pltpu.make_async_copy(
                    msp.at[mq.at[b, pl.ds(16 * s, 16)]],
                    mbuf.at[pl.ds(mb * 16, 16)], msems[mb]).wait()
                srows_v = iota + mb * 16
                rowv, colb = move_rowcol(16 * s)
                col0 = mr[b, pl.ds(16 * s, 16)] * 16

                @pl.loop(0, 16, unroll=4)
                def _(d):
                    vals = plsc.load_gather(mbuf, [srows_v, col0 + d])
                    oscatter(b, rowv, colb + d, vals)

            pltpu.make_async_copy(tbl_h.at[pq.at[b]], pbuf, psem).wait()
            for c in range(3):
                srows_v = iota + 16 * c
                rowv, colb = slot_rowcol(c)
                col0 = pr[b, pl.ds(16 * c, 16)] * 32

                @pl.loop(0, 32, unroll=4)
                def _(d):
                    vals = plsc.load_gather(pbuf, [srows_v, col0 + d])
                    oscatter(b, rowv, colb + d, vals)

            pltpu.async_copy(msp.at[iq.at[b]], pbuf.at[pl.ds(0, Q)], psem)

            for c in range(3):
                rowv, colb = slot_rowcol(c)
                colb = colb + 112
                tflat = tg[pl.ds(b * Q + 16 * c, 16)] * 8

                @pl.loop(0, 8, unroll=4)
                def _(d):
                    fl = tflat + d
                    vals = plsc.load_gather(
                        ttab, [lax.shift_right_logical(fl, 7),
                               lax.bitwise_and(fl, 127)])
                    oscatter(b, rowv, colb + d, vals)

            pltpu.make_async_copy(msp.at[iq.at[b]], pbuf.at[pl.ds(0, Q)],
                                  psem).wait()
            for c in range(3):
                srows_v = iota + 16 * c
                rowv, colb = slot_rowcol(c)
                colb = colb + 96
                col0 = ir[b, pl.ds(16 * c, 16)] * 16

                @pl.loop(0, 16, unroll=4)
                def _(d):
                    vals = plsc.load_gather(pbuf, [srows_v, col0 + d])
                    oscatter(b, rowv, colb + d, vals)

            pltpu.async_copy(obuf.at[pl.ds(b * RPS, RPS)], out_rows(k),
                             wsems[b])

    pltpu.make_async_copy(obuf.at[pl.ds(0, RPS)], out_rows(NQ - 2),
                          wsems[0]).wait()
    pltpu.make_async_copy(obuf.at[pl.ds(RPS, RPS)], out_rows(NQ - 1),
                          wsems[1]).wait()


def kernel(pokemon_ids, move_ids, item_ids, tera_ids, P, M, I, T):
    ids = jnp.concatenate([
        pokemon_ids.astype(jnp.int32).reshape(SLOTS),
        move_ids.astype(jnp.int32).reshape(SLOTS * 4),
        item_ids.astype(jnp.int32).reshape(SLOTS),
        tera_ids.astype(jnp.int32).reshape(SLOTS),
    ])
    tbl = jnp.concatenate([
        P.reshape(PROWS, 128),
        jnp.pad(M.reshape(12500, 128), ((0, MROWS - 12500), (0, 0))),
        jnp.pad(I.reshape(125, 128), ((0, SROWS - IROW0 - 125), (0, 0))),
        jnp.pad(T.reshape(1, 160), ((0, 0), (0, 96))).reshape(2, 128),
    ])
    return _encode(ids, tbl)


# speedup vs baseline: 1.0476x; 1.0029x over previous
import functools

import jax
import jax.numpy as jnp
from jax import lax
from jax.experimental import pallas as pl
from jax.experimental.pallas import tpu as pltpu
from jax.experimental.pallas import tpu_sc as plsc

B = 16384
NSLOT = 6
SLOTS = B * NSLOT
NC, NS = 2, 16
NW = NC * NS
RPS = 8
Q = RPS * NSLOT
SPW = SLOTS // NW
BPW = B // NW
NQ = BPW // RPS
ROW = 120
MROWS = 12544
IROW0 = MROWS
SROWS = 12672
PROWS = 25000
TROW0 = PROWS + SROWS
TBLROWS = TROW0 + 2

_mesh = plsc.VectorSubcoreMesh(
    core_axis_name="c", subcore_axis_name="s", num_cores=NC, num_subcores=NS
)


@functools.partial(
    pl.kernel,
    out_type=jax.ShapeDtypeStruct((B, NSLOT * ROW), jnp.float32),
    mesh=_mesh,
    scratch_types=[
        pltpu.VMEM_SHARED((SROWS, 128), jnp.float32),
        pltpu.VMEM((2, 128), jnp.float32),
        pltpu.VMEM((2 * Q,), jnp.int32),
        pltpu.VMEM((2 * 4 * Q,), jnp.int32),
        pltpu.VMEM((2 * Q,), jnp.int32),
        pltpu.VMEM((2 * Q,), jnp.int32),
        pltpu.VMEM((2, Q), jnp.int32),
        pltpu.VMEM((2, Q), jnp.int32),
        pltpu.VMEM((2, 4 * Q), jnp.int32),
        pltpu.VMEM((2, 4 * Q), jnp.int32),
        pltpu.VMEM((2, Q), jnp.int32),
        pltpu.VMEM((2, Q), jnp.int32),
        pltpu.VMEM((Q, 128), jnp.float32),
        pltpu.VMEM((4 * 16, 128), jnp.float32),
        pltpu.VMEM((2 * RPS, NSLOT * ROW), jnp.float32),
        pltpu.SemaphoreType.DMA,
        pltpu.SemaphoreType.DMA,
        pltpu.SemaphoreType.DMA,
        pltpu.SemaphoreType.DMA,
        pltpu.SemaphoreType.DMA,
        pltpu.SemaphoreType.DMA,
        pltpu.SemaphoreType.DMA,
        pltpu.SemaphoreType.DMA,
        pltpu.SemaphoreType.DMA,
    ],
    compiler_params=pltpu.CompilerParams(needs_layout_passes=False),
)
def _encode(ids_h, tbl_h, out_h,
            msp, ttab, pg, mg, ig, tg, pq, pr, mq, mr, iq, ir,
            pbuf, mbuf, obuf,
            isem0, isem1, psem, msem0, msem1, msem2, msem3,
            wsem0, wsem1):
    sid = lax.axis_index("s")
    cid = lax.axis_index("c")
    wid = sid * NC + cid
    isems = (isem0, isem1)
    msems = (msem0, msem1, msem2, msem3)
    wsems = (wsem0, wsem1)

    srows = SROWS // NS
    pltpu.sync_copy(tbl_h.at[pl.ds(PROWS + sid * srows, srows)],
                    msp.at[pl.ds(sid * srows, srows)])
    pltpu.sync_copy(tbl_h.at[pl.ds(TROW0, 2)], ttab)
    plsc.subcore_barrier()

    iota = lax.iota(jnp.int32, 16)

    def div6(v):
        return lax.shift_right_logical(v * 43691, 18)

    def slot_rowcol(c):
        l = iota + 16 * c
        lb = div6(l)
        return lb, (l - lb * NSLOT) * ROW

    def move_rowcol(fm0):
        fm = iota + fm0
        ml = lax.shift_right_logical(fm, 2)
        mlb = div6(ml)
        return mlb, ((ml - mlb * NSLOT) * ROW + 32
                     + lax.bitwise_and(fm, 3) * 16)

    def idx_list(k, b):
        base = wid * SPW + k * Q
        return [
            (ids_h.at[pl.ds(base, Q)], pg.at[pl.ds(b * Q, Q)]),
            (ids_h.at[pl.ds(SLOTS + 4 * base, 4 * Q)],
             mg.at[pl.ds(b * 4 * Q, 4 * Q)]),
            (ids_h.at[pl.ds(5 * SLOTS + base, Q)], ig.at[pl.ds(b * Q, Q)]),
            (ids_h.at[pl.ds(6 * SLOTS + base, Q)], tg.at[pl.ds(b * Q, Q)]),
        ]

    def stage_idx(k, b):
        for src, dst in idx_list(k, b):
            pltpu.async_copy(src, dst, isems[b])

    def wait_idx(k, b):
        for src, dst in idx_list(k, b):
            pltpu.make_async_copy(src, dst, isems[b]).wait()

    def splits(b):
        for c in range(Q // 16):
            sl = pl.ds(16 * c, 16)
            pv = pg[pl.ds(b * Q + 16 * c, 16)]
            pq[b, sl] = lax.shift_right_logical(pv, 2)
            pr[b, sl] = lax.bitwise_and(pv, 3)
            iv = ig[pl.ds(b * Q + 16 * c, 16)]
            iq[b, sl] = IROW0 + lax.shift_right_logical(iv, 3)
            ir[b, sl] = lax.bitwise_and(iv, 7)
        for c in range(4 * Q // 16):
            sl = pl.ds(16 * c, 16)
            mv = mg[pl.ds(b * 4 * Q + 16 * c, 16)]
            mq[b, sl] = lax.shift_right_logical(mv, 3)
            mr[b, sl] = lax.bitwise_and(mv, 7)

    def out_rows(k):
        return out_h.at[pl.ds(wid * BPW + k * RPS, RPS)]

    def oscatter(b, rowv, colv, vals):
        plsc.store_scatter(obuf, [rowv + b * RPS, colv], vals)

    stage_idx(0, 0)

    @pl.loop(0, NQ, step=2)
    def _(k0):
        for bb in range(2):
            k = k0 + bb
            b, b2 = bb, 1 - bb

            wait_idx(k, b)
            splits(b)

            @pl.when(k + 1 < NQ)
            def _():
                stage_idx(k + 1, b2)

            pltpu.async_copy(tbl_h.at[pq.at[b]], pbuf, psem)
            for s0 in range(3):
                pltpu.async_copy(msp.at[mq.at[b, pl.ds(16 * s0, 16)]],
                                 mbuf.at[pl.ds(16 * s0, 16)], msems[s0])

            @pl.when(k >= 2)
            def _():
                pltpu.make_async_copy(
                    obuf.at[pl.ds(b * RPS, RPS)], out_rows(k - 2),
                    wsems[b]).wait()

            for s in range(12):
                mb = s % 4
                if s + 3 < 12:
                    pltpu.async_copy(
                        msp.at[mq.at[b, pl.ds(16 * (s + 3), 16)]],
                        mbuf.at[pl.ds(((s + 3) % 4) * 16, 16)],
                        msems[(s + 3) % 4])
                pltpu.make_async_copy(
                    msp.at[mq.at[b, pl.ds(16 * s, 16)]],
                    mbuf.at[pl.ds(mb * 16, 16)], msems[mb]).wait()
                srows_v = iota + mb * 16
                rowv, colb = move_rowcol(16 * s)
                col0 = mr[b, pl.ds(16 * s, 16)] * 16

                @pl.loop(0, 16, unroll=4)
                def _(d):
                    vals = plsc.load_gather(mbuf, [srows_v, col0 + d])
                    oscatter(b, rowv, colb + d, vals)

            pltpu.make_async_copy(tbl_h.at[pq.at[b]], pbuf, psem).wait()
            for c in range(3):
                srows_v = iota + 16 * c
                rowv, colb = slot_rowcol(c)
                col0 = pr[b, pl.ds(16 * c, 16)] * 32

                @pl.loop(0, 32, unroll=4)
                def _(d):
                    vals = plsc.load_gather(pbuf, [srows_v, col0 + d])
                    oscatter(b, rowv, colb + d, vals)

            pltpu.async_copy(msp.at[iq.at[b]], pbuf.at[pl.ds(0, Q)], psem)

            for c in range(3):
                rowv, colb = slot_rowcol(c)
                colb = colb + 112
                tflat = tg[pl.ds(b * Q + 16 * c, 16)] * 8

                @pl.loop(0, 8, unroll=4)
                def _(d):
                    fl = tflat + d
                    vals = plsc.load_gather(
                        ttab, [lax.shift_right_logical(fl, 7),
                               lax.bitwise_and(fl, 127)])
                    oscatter(b, rowv, colb + d, vals)

            pltpu.make_async_copy(msp.at[iq.at[b]], pbuf.at[pl.ds(0, Q)],
                                  psem).wait()
            for c in range(3):
                srows_v = iota + 16 * c
                rowv, colb = slot_rowcol(c)
                colb = colb + 96
                col0 = ir[b, pl.ds(16 * c, 16)] * 16

                @pl.loop(0, 16, unroll=4)
                def _(d):
                    vals = plsc.load_gather(pbuf, [srows_v, col0 + d])
                    oscatter(b, rowv, colb + d, vals)

            pltpu.async_copy(obuf.at[pl.ds(b * RPS, RPS)], out_rows(k),
                             wsems[b])

    pltpu.make_async_copy(obuf.at[pl.ds(0, RPS)], out_rows(NQ - 2),
                          wsems[0]).wait()
    pltpu.make_async_copy(obuf.at[pl.ds(RPS, RPS)], out_rows(NQ - 1),
                          wsems[1]).wait()


def kernel(pokemon_ids, move_ids, item_ids, tera_ids, P, M, I, T):
    ids = jnp.concatenate([
        pokemon_ids.astype(jnp.int32).reshape(SLOTS),
        move_ids.astype(jnp.int32).reshape(SLOTS * 4),
        item_ids.astype(jnp.int32).reshape(SLOTS),
        tera_ids.astype(jnp.int32).reshape(SLOTS),
    ])
    tbl = jnp.concatenate([
        P.reshape(PROWS, 128),
        jnp.pad(M.reshape(12500, 128), ((0, MROWS - 12500), (0, 0))),
        jnp.pad(I.reshape(125, 128), ((0, SROWS - IROW0 - 125), (0, 0))),
        jnp.pad(T.reshape(1, 160), ((0, 0), (0, 96))).reshape(2, 128),
    ])
    return _encode(ids, tbl)
